# baseline (speedup 1.0000x reference)
import jax
import jax.numpy as jnp
from jax import lax
from jax.experimental import pallas as pl
from jax.experimental.pallas import tpu as pltpu
from jax.experimental.pallas import tpu_sc as plsc

_NOISE = 0.001
_SCALE = -1.0 / _NOISE

_NB = 4
_NPER = 16 * 512 * 512
_NW = 32
_SPAN = _NPER // _NW
_CH = 8192
_NCHUNK = _SPAN // _CH
_L = 16


def _sc_body(x_hbm, b_hbm, out_hbm, xbuf, bbuf, pbuf):
    cid = lax.axis_index("c")
    sid = lax.axis_index("s")
    wid = sid * 2 + cid

    for b in range(_NB):
        base = b * _NPER + wid * _SPAN

        def chunk_body(c, acc):
            off = base + c * _CH
            pltpu.sync_copy(x_hbm.at[pl.ds(off, _CH)], xbuf)
            pltpu.sync_copy(b_hbm.at[pl.ds(off, _CH)], bbuf)

            def vec_body(i, acc2):
                xv = xbuf[pl.ds(i * _L, _L)]
                bv = bbuf[pl.ds(i * _L, _L)]
                m = jnp.abs(bv) < jnp.float32(jnp.inf)
                d = jnp.where(m, bv - xv, jnp.float32(0.0))
                return acc2 + d * d

            return lax.fori_loop(0, _CH // _L, vec_body, acc)

        acc = lax.fori_loop(0, _NCHUNK, chunk_body,
                            jnp.zeros((_L,), jnp.float32))
        pbuf[...] = acc
        pltpu.sync_copy(pbuf, out_hbm.at[b, wid])


def kernel(x, batch):
    xf = x.reshape(-1)
    bf = batch.reshape(-1)
    mesh = plsc.VectorSubcoreMesh(core_axis_name="c", subcore_axis_name="s")
    k = pl.kernel(
        _sc_body,
        mesh=mesh,
        out_type=jax.ShapeDtypeStruct((_NB, _NW, _L), jnp.float32),
        scratch_types=[
            pltpu.VMEM((_CH,), jnp.float32),
            pltpu.VMEM((_CH,), jnp.float32),
            pltpu.VMEM((_L,), jnp.float32),
        ],
    )
    partial = k(xf, bf)
    return _SCALE * jnp.sum(partial, axis=(1, 2))


# speedup vs baseline: 1.7281x; 1.7281x over previous
import jax
import jax.numpy as jnp
from jax import lax
from jax.experimental import pallas as pl
from jax.experimental.pallas import tpu as pltpu
from jax.experimental.pallas import tpu_sc as plsc

_NOISE = 0.001
_SCALE = -1.0 / _NOISE

_NB = 4
_NPER = 16 * 512 * 512
_NW = 32
_SPAN = _NPER // _NW
_CH = 16384
_NCHUNK = _SPAN // _CH
_L = 16
_U = 4


def _chunk_sum(xbuf, bbuf, par, acc):
    def vec_body(i, accs):
        new = []
        for u in range(_U):
            off = (i * _U + u) * _L
            xv = xbuf[par, pl.ds(off, _L)]
            bv = bbuf[par, pl.ds(off, _L)]
            m = jnp.abs(bv) < jnp.float32(jnp.inf)
            d = jnp.where(m, bv - xv, jnp.float32(0.0))
            new.append(accs[u] + d * d)
        return tuple(new)

    return lax.fori_loop(0, _CH // (_L * _U), vec_body, acc)


def _sc_body(x_hbm, b_hbm, out_hbm, xbuf, bbuf, pbuf, sx0, sx1, sb0, sb1):
    cid = lax.axis_index("c")
    sid = lax.axis_index("s")
    wid = sid * 2 + cid
    sems = ((sx0, sb0), (sx1, sb1))

    for b in range(_NB):
        base = b * _NPER + wid * _SPAN

        waits = [None, None]

        def issue(c):
            par = c % 2
            sx, sb = sems[par]
            hx = pltpu.async_copy(
                x_hbm.at[pl.ds(base + c * _CH, _CH)], xbuf.at[par], sx)
            hb = pltpu.async_copy(
                b_hbm.at[pl.ds(base + c * _CH, _CH)], bbuf.at[par], sb)
            waits[par] = (hx, hb)

        issue(0)
        issue(1)
        acc = tuple(jnp.zeros((_L,), jnp.float32) for _ in range(_U))
        for c in range(_NCHUNK):
            par = c % 2
            hx, hb = waits[par]
            hx.wait()
            hb.wait()
            acc = _chunk_sum(xbuf, bbuf, par, acc)
            if c + 2 < _NCHUNK:
                issue(c + 2)

        total = (acc[0] + acc[1]) + (acc[2] + acc[3])
        pbuf[...] = total
        pltpu.sync_copy(pbuf, out_hbm.at[b, wid])


def kernel(x, batch):
    xf = x.reshape(-1)
    bf = batch.reshape(-1)
    mesh = plsc.VectorSubcoreMesh(core_axis_name="c", subcore_axis_name="s")
    k = pl.kernel(
        _sc_body,
        mesh=mesh,
        out_type=jax.ShapeDtypeStruct((_NB, _NW, _L), jnp.float32),
        scratch_types=[
            pltpu.VMEM((2, _CH), jnp.float32),
            pltpu.VMEM((2, _CH), jnp.float32),
            pltpu.VMEM((_L,), jnp.float32),
            pltpu.SemaphoreType.DMA,
            pltpu.SemaphoreType.DMA,
            pltpu.SemaphoreType.DMA,
            pltpu.SemaphoreType.DMA,
        ],
    )
    partial = k(xf, bf)
    return _SCALE * jnp.sum(partial, axis=(1, 2))
